# baseline (speedup 1.0000x reference)
import jax
import jax.numpy as jnp
from jax import lax
from jax.experimental import pallas as pl
from jax.experimental.pallas import tpu as pltpu
from jax.experimental.pallas import tpu_sc as plsc

B = 16384
D = 32
NC = 2
NS = 16
NW = NC * NS
BPW = B // NW
GROUPS = BPW // 16
BLK = 128


def _mf_body(uids_hbm, iids_hbm, uT_hbm, iT_hbm, ubp_hbm, ibp_hbm,
             out_hbm,
             uid_v, iid_v, ub_v, ib_v, out_v,
             ublks, iblks,
             sem_u, sem_i, sem_b):
    wid = lax.axis_index("s") * NC + lax.axis_index("c")
    base = wid * BPW

    pltpu.sync_copy(uids_hbm.at[pl.ds(base, BPW)], uid_v)
    pltpu.sync_copy(iids_hbm.at[pl.ds(base, BPW)], iid_v)
    cp_ub = pltpu.async_copy(ubp_hbm.at[uid_v], ub_v, sem_b)
    cp_ib = pltpu.async_copy(ibp_hbm.at[iid_v], ib_v, sem_b)

    lanes = lax.iota(jnp.int32, 16)
    zero16 = jnp.zeros((16,), jnp.int32)

    def extract(j):
        off = pl.multiple_of((j >> 4) * 16, 16)
        m = lanes == (j & 15)
        uvec = uid_v[pl.ds(off, 16)]
        ivec = iid_v[pl.ds(off, 16)]
        us = lax.reduce_sum_p.bind(jnp.where(m, uvec, zero16), axes=(0,))
        vs = lax.reduce_sum_p.bind(jnp.where(m, ivec, zero16), axes=(0,))
        return us, vs

    def fetch(us, vs, ublk, iblk):
        ub = pl.multiple_of((us >> 7) * BLK, BLK)
        ib = pl.multiple_of((vs >> 7) * BLK, BLK)
        pltpu.async_copy(uT_hbm.at[:, pl.ds(ub, BLK)], ublk, sem_u)
        pltpu.async_copy(iT_hbm.at[:, pl.ds(ib, BLK)], iblk, sem_i)

    def consume(us, vs, ublk, iblk):
        pltpu.make_async_copy(uT_hbm.at[:, pl.ds(0, BLK)], ublk, sem_u).wait()
        pltpu.make_async_copy(iT_hbm.at[:, pl.ds(0, BLK)], iblk, sem_i).wait()
        cu = jnp.broadcast_to(us & (BLK - 1), (16,)).astype(jnp.int32)
        ci = jnp.broadcast_to(vs & (BLK - 1), (16,)).astype(jnp.int32)
        u0 = plsc.load_gather(ublk, [lanes, cu])
        u1 = plsc.load_gather(ublk, [lanes + 16, cu])
        i0 = plsc.load_gather(iblk, [lanes, ci])
        i1 = plsc.load_gather(iblk, [lanes + 16, ci])
        return lax.reduce_sum_p.bind(u0 * i0 + u1 * i1, axes=(0,))

    bufs = list(zip(ublks, iblks))
    DEPTH = 12
    pend = []
    for j in range(DEPTH - 1):
        e = extract(j)
        fetch(e[0], e[1], *bufs[j])
        pend.append(e)
    cp_ub.wait()
    cp_ib.wait()

    def group(g, carry):
        pend = list(zip(carry[0::2], carry[1::2]))
        j0 = g * 16
        sl = pl.ds(j0, 16)
        acc = ub_v[sl] + ib_v[sl]
        for k in range(16):
            j = j0 + k
            jn = jnp.minimum(j + DEPTH - 1, BPW - 1)
            en = extract(jn)

            @pl.when(j + DEPTH - 1 < BPW)
            def _():
                fetch(en[0], en[1], *bufs[(k + DEPTH - 1) % DEPTH])

            us, vs = pend[0]
            s = consume(us, vs, *bufs[k % DEPTH])
            acc = jnp.where(lanes == k, jnp.broadcast_to(s, (16,)), acc)
            pend = pend[1:] + [en]
        out_v[sl] = acc
        return tuple(x for e in pend for x in e)

    init = tuple(x for e in pend for x in e)
    lax.fori_loop(0, GROUPS, group, init)
    pltpu.sync_copy(out_v, out_hbm.at[pl.ds(base, BPW)])


def kernel(users_ids, items_ids, user_bias, item_bias, user_emb_table,
           item_emb_table, global_bias, user_bias_param, item_bias_param):
    mesh = plsc.VectorSubcoreMesh(core_axis_name="c", subcore_axis_name="s",
                                  num_cores=NC, num_subcores=NS)
    run = pl.kernel(
        _mf_body,
        out_type=jax.ShapeDtypeStruct((B,), jnp.float32),
        mesh=mesh,
        compiler_params=pltpu.CompilerParams(needs_layout_passes=False,
                                             use_tc_tiling_on_sc=True),
        scratch_types=[
            pltpu.VMEM((BPW,), jnp.int32),
            pltpu.VMEM((BPW,), jnp.int32),
            pltpu.VMEM((BPW,), jnp.float32),
            pltpu.VMEM((BPW,), jnp.float32),
            pltpu.VMEM((BPW,), jnp.float32),
            [pltpu.VMEM((D, BLK), jnp.float32) for _ in range(12)],
            [pltpu.VMEM((D, BLK), jnp.float32) for _ in range(12)],
            pltpu.SemaphoreType.DMA,
            pltpu.SemaphoreType.DMA,
            pltpu.SemaphoreType.DMA,
        ],
    )
    pred = run(users_ids.astype(jnp.int32), items_ids.astype(jnp.int32),
               user_emb_table.T, item_emb_table.T,
               user_bias_param, item_bias_param)
    return pred + global_bias


# speedup vs baseline: 1.0000x; 1.0000x over previous
import jax
import jax.numpy as jnp
from jax import lax
from jax.experimental import pallas as pl
from jax.experimental.pallas import tpu as pltpu
from jax.experimental.pallas import tpu_sc as plsc

B = 16384
D = 32
NC = 2
NS = 16
NW = NC * NS
BPW = B // NW
GROUPS = BPW // 16
BLK = 128


def _mf_body(uids_hbm, iids_hbm, uT_hbm, iT_hbm, ubp_hbm, ibp_hbm,
             out_hbm,
             uid_v, iid_v, ub_v, ib_v, out_v,
             ublks, iblks,
             sem_u, sem_i, sem_b):
    wid = lax.axis_index("s") * NC + lax.axis_index("c")
    base = wid * BPW

    pltpu.sync_copy(uids_hbm.at[pl.ds(base, BPW)], uid_v)
    pltpu.sync_copy(iids_hbm.at[pl.ds(base, BPW)], iid_v)
    cp_ub = pltpu.async_copy(ubp_hbm.at[uid_v], ub_v, sem_b)
    cp_ib = pltpu.async_copy(ibp_hbm.at[iid_v], ib_v, sem_b)

    lanes = lax.iota(jnp.int32, 16)
    zero16 = jnp.zeros((16,), jnp.int32)

    def extract(j):
        off = pl.multiple_of((j >> 4) * 16, 16)
        m = lanes == (j & 15)
        uvec = uid_v[pl.ds(off, 16)]
        ivec = iid_v[pl.ds(off, 16)]
        us = lax.reduce_sum_p.bind(jnp.where(m, uvec, zero16), axes=(0,))
        vs = lax.reduce_sum_p.bind(jnp.where(m, ivec, zero16), axes=(0,))
        return us, vs

    def fetch(us, vs, ublk, iblk):
        ub = pl.multiple_of((us >> 7) * BLK, BLK)
        ib = pl.multiple_of((vs >> 7) * BLK, BLK)
        pltpu.async_copy(uT_hbm.at[:, pl.ds(ub, BLK)], ublk, sem_u)
        pltpu.async_copy(iT_hbm.at[:, pl.ds(ib, BLK)], iblk, sem_i)

    def consume(us, vs, ublk, iblk):
        pltpu.make_async_copy(uT_hbm.at[:, pl.ds(0, BLK)], ublk, sem_u).wait()
        pltpu.make_async_copy(iT_hbm.at[:, pl.ds(0, BLK)], iblk, sem_i).wait()
        cu = jnp.broadcast_to(us & (BLK - 1), (16,)).astype(jnp.int32)
        ci = jnp.broadcast_to(vs & (BLK - 1), (16,)).astype(jnp.int32)
        u0 = plsc.load_gather(ublk, [lanes, cu])
        u1 = plsc.load_gather(ublk, [lanes + 16, cu])
        i0 = plsc.load_gather(iblk, [lanes, ci])
        i1 = plsc.load_gather(iblk, [lanes + 16, ci])
        return lax.reduce_sum_p.bind(u0 * i0 + u1 * i1, axes=(0,))

    bufs = list(zip(ublks, iblks))
    DEPTH = 8
    pend = []
    for j in range(DEPTH - 1):
        e = extract(j)
        fetch(e[0], e[1], *bufs[j])
        pend.append(e)
    cp_ub.wait()
    cp_ib.wait()

    def group(g, carry):
        pend = list(zip(carry[0::2], carry[1::2]))
        j0 = g * 16
        sl = pl.ds(j0, 16)
        acc = ub_v[sl] + ib_v[sl]
        for k in range(16):
            j = j0 + k
            jn = jnp.minimum(j + DEPTH - 1, BPW - 1)
            en = extract(jn)

            @pl.when(j + DEPTH - 1 < BPW)
            def _():
                fetch(en[0], en[1], *bufs[(k + DEPTH - 1) % DEPTH])

            us, vs = pend[0]
            s = consume(us, vs, *bufs[k % DEPTH])
            acc = jnp.where(lanes == k, jnp.broadcast_to(s, (16,)), acc)
            pend = pend[1:] + [en]
        out_v[sl] = acc
        return tuple(x for e in pend for x in e)

    init = tuple(x for e in pend for x in e)
    lax.fori_loop(0, GROUPS, group, init)
    pltpu.sync_copy(out_v, out_hbm.at[pl.ds(base, BPW)])


def kernel(users_ids, items_ids, user_bias, item_bias, user_emb_table,
           item_emb_table, global_bias, user_bias_param, item_bias_param):
    mesh = plsc.VectorSubcoreMesh(core_axis_name="c", subcore_axis_name="s",
                                  num_cores=NC, num_subcores=NS)
    run = pl.kernel(
        _mf_body,
        out_type=jax.ShapeDtypeStruct((B,), jnp.float32),
        mesh=mesh,
        compiler_params=pltpu.CompilerParams(needs_layout_passes=False,
                                             use_tc_tiling_on_sc=True),
        scratch_types=[
            pltpu.VMEM((BPW,), jnp.int32),
            pltpu.VMEM((BPW,), jnp.int32),
            pltpu.VMEM((BPW,), jnp.float32),
            pltpu.VMEM((BPW,), jnp.float32),
            pltpu.VMEM((BPW,), jnp.float32),
            [pltpu.VMEM((D, BLK), jnp.float32) for _ in range(8)],
            [pltpu.VMEM((D, BLK), jnp.float32) for _ in range(8)],
            pltpu.SemaphoreType.DMA,
            pltpu.SemaphoreType.DMA,
            pltpu.SemaphoreType.DMA,
        ],
    )
    pred = run(users_ids.astype(jnp.int32), items_ids.astype(jnp.int32),
               user_emb_table.T, item_emb_table.T,
               user_bias_param, item_bias_param)
    return pred + global_bias
